# baseline (speedup 1.0000x reference)
import jax
import jax.numpy as jnp
from jax.experimental import pallas as pl
from jax.experimental.pallas import tpu as pltpu

_N = 4096
_DIN = 512
_DH = 512
_DOUT = 256
_BLK = 256
_NBLK = _N // _BLK
_TCH = 512
_NTCH = _N // _TCH
_EPS = 1e-5
_INV_N = 1.0 / _N


def _fold128(a):
    a128 = a[0:128, :] + a[128:256, :]
    a64 = a128[0:64, :] + a128[64:128, :]
    a32 = a64[0:32, :] + a64[32:64, :]
    a16 = a32[0:16, :] + a32[16:32, :]
    a8 = a16[0:8, :] + a16[8:16, :]
    a4 = a8[0:4, :] + a8[4:8, :]
    a2 = a4[0:2, :] + a4[2:4, :]
    return (a2[0:1, :] + a2[1:2, :])[0]


def _gcn_body(x_ref, adj_ref, w_ref, p_ref, out_ref, h_ref, t_ref, acc_ref):
    p = pl.program_id(0)
    i = pl.program_id(1)

    @pl.when(p == 0)
    def _t_first():
        t_ref[pl.ds(i * _BLK, _BLK), :] = jnp.dot(
            x_ref[0], w_ref[0], preferred_element_type=jnp.float32)

    @pl.when(jnp.logical_and(p >= 2, i == 0))
    def _bn_start():
        z256 = jnp.zeros((256, _DH), jnp.float32)

        def _macc(k, a):
            return a + h_ref[pl.ds(k * 256, 256), :]

        a256 = jax.lax.fori_loop(0, _N // 256, _macc, z256)
        mean = _fold128(a256) * _INV_N

        def _sacc(k, a):
            d = h_ref[pl.ds(k * 256, 256), :] - mean[None, :]
            return a + d * d

        s256 = jax.lax.fori_loop(0, _N // 256, _sacc, z256)
        sq = jnp.sqrt(_fold128(s256) * _INV_N + _EPS)
        acc_ref[0, :] = mean
        acc_ref[1, :] = sq

        @pl.when(p <= 3)
        def _t_rest():
            g = p_ref[0, 1, :]
            beta = p_ref[0, 2, :]

            def _chunk(k, carry):
                hc = h_ref[pl.ds(k * _TCH, _TCH), :]
                hn = ((hc - mean[None, :]) / sq[None, :] * g[None, :]
                      + beta[None, :])
                t_ref[pl.ds(k * _TCH, _TCH), :] = jnp.dot(
                    hn, w_ref[0], preferred_element_type=jnp.float32)
                return carry

            jax.lax.fori_loop(0, _NTCH, _chunk, 0)

    @pl.when(jnp.logical_and(p >= 1, p <= 3))
    def _layer():
        y = jnp.dot(adj_ref[0], t_ref[:], preferred_element_type=jnp.float32)
        y = jnp.maximum(y + p_ref[0, 0, :][None, :], 0.0)
        h_ref[pl.ds(i * _BLK, _BLK), :] = y
        out_ref[:] = y[:, :_DOUT]

    @pl.when(p == 4)
    def _final_bn():
        y = h_ref[pl.ds(i * _BLK, _BLK), :]
        g = p_ref[0, 1, :]
        beta = p_ref[0, 2, :]
        yn = ((y - acc_ref[0, :][None, :]) / acc_ref[1, :][None, :]
              * g[None, :] + beta[None, :])
        out_ref[:] = yn[:, :_DOUT]


def kernel(x, adj, W1, b1, W2, b2, W3, b3, g1, beta1, g2, beta2, g3, beta3):
    w3p = jnp.pad(W3, ((0, 0), (0, _DH - _DOUT)))
    zw = jnp.zeros((_DIN, _DH), jnp.float32)
    w_all = jnp.stack([W1, zw, W2, w3p, zw])

    def _pad(v):
        return jnp.pad(v, (0, _DH - v.shape[0]))

    z = jnp.zeros((_DH,), jnp.float32)
    p_all = jnp.stack([
        jnp.stack([z, z, z] + [z] * 5),
        jnp.stack([b1, z, z] + [z] * 5),
        jnp.stack([b2, g1, beta1] + [z] * 5),
        jnp.stack([_pad(b3), g2, beta2] + [z] * 5),
        jnp.stack([z, _pad(g3), _pad(beta3)] + [z] * 5),
    ])

    return pl.pallas_call(
        _gcn_body,
        grid=(5, _NBLK),
        in_specs=[
            pl.BlockSpec((1, _BLK, _DIN),
                         lambda p, i: (0, jnp.where(p == 0, i, 0), 0)),
            pl.BlockSpec(
                (1, _BLK, _N),
                lambda p, i: (0, jnp.where(jnp.logical_and(p >= 1, p <= 3),
                                           i, 0), 0)),
            pl.BlockSpec((1, _DIN, _DH), lambda p, i: (p, 0, 0)),
            pl.BlockSpec((1, 8, _DH), lambda p, i: (p, 0, 0)),
        ],
        out_specs=pl.BlockSpec((_BLK, _DOUT), lambda p, i: (i, 0)),
        out_shape=jax.ShapeDtypeStruct((_N, _DOUT), jnp.float32),
        scratch_shapes=[
            pltpu.VMEM((_N, _DH), jnp.float32),
            pltpu.VMEM((_N, _DH), jnp.float32),
            pltpu.VMEM((8, _DH), jnp.float32),
        ],
        compiler_params=pltpu.CompilerParams(
            dimension_semantics=("arbitrary", "arbitrary"),
        ),
    )(x, adj, w_all, p_all)
